# baseline (speedup 1.0000x reference)
import jax
import jax.numpy as jnp
from jax.experimental import pallas as pl
from jax.experimental.pallas import tpu as pltpu

_B, _S, _H, _E, _F, _MAXK = 1, 2048, 1024, 8, 2048, 6
_TT = 256
_NT = _S // _TT
_FC = 1024
_NF = _F // _FC


def _routing_body(x_ref, rw_ref, aw1_ref, ab1_ref, aw2_ref, ab2_ref, w_ref):
    x = x_ref[...]
    rl = jnp.dot(x, rw_ref[...], preferred_element_type=jnp.float32)
    h = jax.nn.gelu(jnp.dot(x, aw1_ref[...], preferred_element_type=jnp.float32)
                    + ab1_ref[...])
    al = jnp.clip(jnp.dot(h, aw2_ref[...], preferred_element_type=jnp.float32)
                  + ab2_ref[...], -30.0, 30.0)
    ap = jnp.clip(jax.nn.softmax(al, axis=-1), 1e-8, 1.0)
    k = (jnp.argmax(ap, axis=-1) + 1).astype(jnp.int32)

    eidx = jax.lax.broadcasted_iota(jnp.int32, (_TT, _E), 1)
    rank = jnp.zeros((_TT, _E), dtype=jnp.int32)
    for j in range(_E):
        lj = rl[:, j:j + 1]
        beats = (lj > rl) | ((lj == rl) & (j < eidx))
        rank = rank + beats.astype(jnp.int32)
    selected = rank < k[:, None]

    masked = jnp.where(selected, rl, jnp.full_like(rl, -1e9))
    m = jnp.max(masked, axis=-1, keepdims=True)
    ew = jnp.exp(masked - m)
    w = ew / jnp.sum(ew, axis=-1, keepdims=True)
    w_ref[...] = jnp.where(selected, w, 0.0).T


def _sparse_expert_body(x_ref, w_ref, gate_ref, up_ref, down_ref, out_ref,
                        counts_ref, ids_ref, wl_ref, xg_ref, pt_ref):
    e = pl.program_id(0)
    f = pl.program_id(1)
    t = pl.program_id(2)

    @pl.when((e == 0) & (f == 0) & (t == 0))
    def _prologue():
        out_ref[...] = jnp.zeros_like(out_ref)
        for ee in range(_E):
            counts_ref[ee] = 0

        def scan(tok, carry):
            for ee in range(_E):
                wv = w_ref[ee, tok]

                @pl.when(wv > 0.0)
                def _append():
                    c = counts_ref[ee]
                    ids_ref[ee, c] = tok
                    wl_ref[ee, c] = wv
                    counts_ref[ee] = c + 1
            return carry

        jax.lax.fori_loop(0, _S, scan, 0)

    cnt = counts_ref[e]
    base = t * _TT

    @pl.when(base < cnt)
    def _compute():
        @pl.when(f == 0)
        def _gather():
            def g_body(i, carry):
                @pl.when(base + i < cnt)
                def _row():
                    tid = ids_ref[e, base + i]
                    xg_ref[pl.ds(base + i, 1), :] = x_ref[pl.ds(tid, 1), :]
                return carry

            jax.lax.fori_loop(0, _TT, g_body, 0)

        xt = xg_ref[pl.ds(base, _TT), :]
        g = jax.nn.silu(jnp.dot(xt, gate_ref[0],
                                preferred_element_type=jnp.float32))
        g = g * jnp.dot(xt, up_ref[0], preferred_element_type=jnp.float32)
        pt_ref[...] = jnp.dot(g, down_ref[0],
                              preferred_element_type=jnp.float32)

        def s_body(i, carry):
            @pl.when(base + i < cnt)
            def _row():
                tid = ids_ref[e, base + i]
                wt = wl_ref[e, base + i]
                out_ref[pl.ds(tid, 1), :] += wt * pt_ref[pl.ds(i, 1), :]
            return carry

        jax.lax.fori_loop(0, _TT, s_body, 0)


def kernel(hidden_states, router_w, actor_w1, actor_b1, actor_w2, actor_b2,
           gate_w, up_w, down_w):
    x2d = hidden_states.reshape(_S, _H)

    w = pl.pallas_call(
        _routing_body,
        grid=(_NT,),
        in_specs=[
            pl.BlockSpec((_TT, _H), lambda t: (t, 0)),
            pl.BlockSpec((_H, _E), lambda t: (0, 0)),
            pl.BlockSpec((_H, _H), lambda t: (0, 0)),
            pl.BlockSpec((1, _H), lambda t: (0, 0)),
            pl.BlockSpec((_H, _MAXK), lambda t: (0, 0)),
            pl.BlockSpec((1, _MAXK), lambda t: (0, 0)),
        ],
        out_specs=pl.BlockSpec((_E, _TT), lambda t: (0, t)),
        out_shape=jax.ShapeDtypeStruct((_E, _S), jnp.float32),
    )(x2d, router_w, actor_w1, actor_b1.reshape(1, _H),
      actor_w2, actor_b2.reshape(1, _MAXK))

    out = pl.pallas_call(
        _sparse_expert_body,
        grid=(_E, _NF, _NT),
        in_specs=[
            pl.BlockSpec((_S, _H), lambda e, f, t: (0, 0)),
            pl.BlockSpec(memory_space=pltpu.SMEM),
            pl.BlockSpec((1, _H, _FC), lambda e, f, t: (e, 0, f)),
            pl.BlockSpec((1, _H, _FC), lambda e, f, t: (e, 0, f)),
            pl.BlockSpec((1, _FC, _H), lambda e, f, t: (e, f, 0)),
        ],
        out_specs=pl.BlockSpec((_S, _H), lambda e, f, t: (0, 0)),
        out_shape=jax.ShapeDtypeStruct((_S, _H), jnp.float32),
        scratch_shapes=[
            pltpu.SMEM((_E,), jnp.int32),
            pltpu.SMEM((_E, _S), jnp.int32),
            pltpu.SMEM((_E, _S), jnp.float32),
            pltpu.VMEM((_S, _H), jnp.float32),
            pltpu.VMEM((_TT, _H), jnp.float32),
        ],
    )(x2d, w, gate_w, up_w, down_w)

    return out.reshape(_B, _S, _H)


# speedup vs baseline: 2.3139x; 2.3139x over previous
import jax
import jax.numpy as jnp
from jax.experimental import pallas as pl
from jax.experimental.pallas import tpu as pltpu

_B, _S, _H, _E, _F, _MAXK = 1, 2048, 1024, 8, 2048, 6
_TT = 256
_NT = _S // _TT
_FC = 1024
_NF = _F // _FC


def _routing_body(x_ref, rw_ref, aw1_ref, ab1_ref, aw2_ref, ab2_ref, w_ref):
    x = x_ref[...]
    rl = jnp.dot(x, rw_ref[...], preferred_element_type=jnp.float32)
    h = jax.nn.gelu(jnp.dot(x, aw1_ref[...], preferred_element_type=jnp.float32)
                    + ab1_ref[...])
    al = jnp.clip(jnp.dot(h, aw2_ref[...], preferred_element_type=jnp.float32)
                  + ab2_ref[...], -30.0, 30.0)
    ap = jnp.clip(jax.nn.softmax(al, axis=-1), 1e-8, 1.0)
    k = (jnp.argmax(ap, axis=-1) + 1).astype(jnp.int32)

    eidx = jax.lax.broadcasted_iota(jnp.int32, (_TT, _E), 1)
    rank = jnp.zeros((_TT, _E), dtype=jnp.int32)
    for j in range(_E):
        lj = rl[:, j:j + 1]
        beats = (lj > rl) | ((lj == rl) & (j < eidx))
        rank = rank + beats.astype(jnp.int32)
    selected = rank < k[:, None]

    masked = jnp.where(selected, rl, jnp.full_like(rl, -1e9))
    m = jnp.max(masked, axis=-1, keepdims=True)
    ew = jnp.exp(masked - m)
    w = ew / jnp.sum(ew, axis=-1, keepdims=True)
    w_ref[...] = jnp.where(selected, w, 0.0)


def _dense_expert_body(x_ref, w_ref, gate_ref, up_ref, down_ref, out_ref):
    e = pl.program_id(0)
    f = pl.program_id(1)
    t = pl.program_id(2)

    @pl.when((e == 0) & (f == 0) & (t == 0))
    def _init():
        out_ref[...] = jnp.zeros_like(out_ref)

    xt = x_ref[pl.ds(t * _TT, _TT), :].astype(jnp.bfloat16)
    g = jax.nn.silu(jnp.dot(xt, gate_ref[0].astype(jnp.bfloat16),
                            preferred_element_type=jnp.float32))
    g = g * jnp.dot(xt, up_ref[0].astype(jnp.bfloat16),
                    preferred_element_type=jnp.float32)
    y = jnp.dot(g.astype(jnp.bfloat16), down_ref[0].astype(jnp.bfloat16),
                preferred_element_type=jnp.float32)

    wblk = w_ref[pl.ds(t * _TT, _TT), :]
    col = jax.lax.broadcasted_iota(jnp.int32, (_TT, _E), 1)
    wt = jnp.sum(jnp.where(col == e, wblk, 0.0), axis=1, keepdims=True)
    out_ref[pl.ds(t * _TT, _TT), :] += wt * y


def kernel(hidden_states, router_w, actor_w1, actor_b1, actor_w2, actor_b2,
           gate_w, up_w, down_w):
    x2d = hidden_states.reshape(_S, _H)

    w = pl.pallas_call(
        _routing_body,
        grid=(_NT,),
        in_specs=[
            pl.BlockSpec((_TT, _H), lambda t: (t, 0)),
            pl.BlockSpec((_H, _E), lambda t: (0, 0)),
            pl.BlockSpec((_H, _H), lambda t: (0, 0)),
            pl.BlockSpec((1, _H), lambda t: (0, 0)),
            pl.BlockSpec((_H, _MAXK), lambda t: (0, 0)),
            pl.BlockSpec((1, _MAXK), lambda t: (0, 0)),
        ],
        out_specs=pl.BlockSpec((_TT, _E), lambda t: (t, 0)),
        out_shape=jax.ShapeDtypeStruct((_S, _E), jnp.float32),
    )(x2d, router_w, actor_w1, actor_b1.reshape(1, _H),
      actor_w2, actor_b2.reshape(1, _MAXK))

    out = pl.pallas_call(
        _dense_expert_body,
        grid=(_E, _NF, _NT),
        in_specs=[
            pl.BlockSpec((_S, _H), lambda e, f, t: (0, 0)),
            pl.BlockSpec((_S, _E), lambda e, f, t: (0, 0)),
            pl.BlockSpec((1, _H, _FC), lambda e, f, t: (e, 0, f)),
            pl.BlockSpec((1, _H, _FC), lambda e, f, t: (e, 0, f)),
            pl.BlockSpec((1, _FC, _H), lambda e, f, t: (e, f, 0)),
        ],
        out_specs=pl.BlockSpec((_S, _H), lambda e, f, t: (0, 0)),
        out_shape=jax.ShapeDtypeStruct((_S, _H), jnp.float32),
    )(x2d, w, gate_w, up_w, down_w)

    return out.reshape(_B, _S, _H)
